# baseline (speedup 1.0000x reference)
import functools

import jax
import jax.numpy as jnp
from jax import lax
from jax.experimental import pallas as pl
from jax.experimental.pallas import tpu as pltpu
from jax.experimental.pallas import tpu_sc as plsc

N = 10000
E = 320000
D = 128
NC = 2
NS = 16
NW = NC * NS
CH = 128
NCHUNK = 80
E2 = NW * NCHUNK * CH
NH = NCHUNK // 2
NPAD = 10240
RPT = NPAD // NS
ZR = 128
DegW = 16
BR = 1000
L = 16


def _fill_const(ref, rows, width, value):
    @pl.loop(0, rows * (width // L))
    def _(i):
        r = i // (width // L)
        c = i % (width // L)
        ref[r, pl.ds(c * L, L)] = jnp.full((L,), value, jnp.float32)


def _deg_body(dst3_hbm, out_hbm, deg_sh, ones_v, idx_v, zbuf_v):
    c = lax.axis_index("c")
    s = lax.axis_index("s")
    wid = c * NS + s
    @pl.loop(0, (CH + L - 1) // L)
    def _(i):
        off = jnp.minimum(i * L, CH - L)
        ones_v[pl.ds(off, L)] = jnp.full((L,), 1.0, jnp.float32)
    @pl.loop(0, RPT // L)
    def _(i):
        zbuf_v[pl.ds(i * L, L)] = jnp.zeros((L,), jnp.float32)
    pltpu.sync_copy(zbuf_v, deg_sh.at[pl.ds(s * RPT, RPT)])
    pltpu.sync_copy(dst3_hbm.at[wid], idx_v)
    plsc.subcore_barrier()
    @pl.loop(0, NCHUNK)
    def _(k):
        pltpu.sync_copy(ones_v, deg_sh.at[idx_v.at[k]], add=True)
    plsc.subcore_barrier()
    pltpu.sync_copy(deg_sh.at[pl.ds(s * RPT, RPT)],
                    out_hbm.at[pl.ds(c * NPAD + s * RPT, RPT)])


def _agg_body(src3_hbm, dst3_hbm, tbl_hbm, out_hbm,
              acc_sh, sidx_v, didx_v, rows0, rows1, sg0, sg1, ss0, ss1):
    c = lax.axis_index("c")
    s = lax.axis_index("s")
    wid = c * NS + s
    _fill_const(rows0, CH, D, 0.0)
    @pl.loop(0, RPT // CH)
    def _(j):
        pltpu.sync_copy(rows0, acc_sh.at[pl.ds(s * RPT + j * CH, CH)])
    plsc.subcore_barrier()
    for h in range(2):
        pltpu.sync_copy(src3_hbm.at[wid, pl.ds(h * NH, NH)], sidx_v)
        pltpu.sync_copy(dst3_hbm.at[wid, pl.ds(h * NH, NH)], didx_v)
        pltpu.async_copy(tbl_hbm.at[sidx_v.at[0]], rows0, sg0)
        pltpu.async_copy(tbl_hbm.at[sidx_v.at[1]], rows1, sg1)
        @pl.loop(0, NH - 2, step=2)
        def _(k):
            pltpu.make_async_copy(tbl_hbm.at[sidx_v.at[k]], rows0, sg0).wait()
            pltpu.async_copy(rows0, acc_sh.at[didx_v.at[k]], ss0, add=True)
            pltpu.make_async_copy(tbl_hbm.at[sidx_v.at[k + 1]], rows1, sg1).wait()
            pltpu.async_copy(rows1, acc_sh.at[didx_v.at[k + 1]], ss1, add=True)
            pltpu.make_async_copy(rows0, acc_sh.at[didx_v.at[k]], ss0).wait()
            pltpu.async_copy(tbl_hbm.at[sidx_v.at[k + 2]], rows0, sg0)
            pltpu.make_async_copy(rows1, acc_sh.at[didx_v.at[k + 1]], ss1).wait()
            pltpu.async_copy(tbl_hbm.at[sidx_v.at[k + 3]], rows1, sg1)
        pltpu.make_async_copy(tbl_hbm.at[sidx_v.at[NH - 2]], rows0, sg0).wait()
        pltpu.async_copy(rows0, acc_sh.at[didx_v.at[NH - 2]], ss0, add=True)
        pltpu.make_async_copy(tbl_hbm.at[sidx_v.at[NH - 1]], rows1, sg1).wait()
        pltpu.async_copy(rows1, acc_sh.at[didx_v.at[NH - 1]], ss1, add=True)
        pltpu.make_async_copy(rows0, acc_sh.at[didx_v.at[NH - 2]], ss0).wait()
        pltpu.make_async_copy(rows1, acc_sh.at[didx_v.at[NH - 1]], ss1).wait()
    plsc.subcore_barrier()
    pltpu.sync_copy(acc_sh.at[pl.ds(s * RPT, RPT)],
                    out_hbm.at[pl.ds(c * NPAD + s * RPT, RPT)])


_SC_MESH = plsc.VectorSubcoreMesh(
    core_axis_name="c", subcore_axis_name="s", num_cores=NC, num_subcores=NS)

_deg_call = pl.kernel(
    _deg_body,
    out_type=jax.ShapeDtypeStruct((NC * NPAD,), jnp.float32),
    mesh=_SC_MESH,
    scratch_types=[
        pltpu.VMEM_SHARED((NPAD,), jnp.float32),
        pltpu.VMEM((CH,), jnp.float32),
        pltpu.VMEM((NCHUNK, CH), jnp.int32),
        pltpu.VMEM((RPT,), jnp.float32),
    ],
)

_agg_call = pl.kernel(
    _agg_body,
    out_type=jax.ShapeDtypeStruct((NC * NPAD, D), jnp.float32),
    mesh=_SC_MESH,
    scratch_types=[
        pltpu.VMEM_SHARED((NPAD, D), jnp.float32),
        pltpu.VMEM((NH, CH), jnp.int32),
        pltpu.VMEM((NH, CH), jnp.int32),
        pltpu.VMEM((CH, D), jnp.float32),
        pltpu.VMEM((CH, D), jnp.float32),
        pltpu.SemaphoreType.DMA,
        pltpu.SemaphoreType.DMA,
        pltpu.SemaphoreType.DMA,
        pltpu.SemaphoreType.DMA,
    ],
)


def _tc1_body(deg_ref, x_ref, w_ref, hs_ref, dinv_ref):
    deg = deg_ref[0] + deg_ref[1] + 1.0
    dinv1 = lax.rsqrt(deg)
    h = jnp.dot(x_ref[...], w_ref[...], preferred_element_type=jnp.float32)
    hs_ref[...] = h * dinv1
    dinv_ref[...] = jnp.broadcast_to(dinv1, (BR, D))


_tc1_call = pl.pallas_call(
    _tc1_body,
    grid=(N // BR,),
    in_specs=[
        pl.BlockSpec((NC, BR, 1), lambda i: (0, i, 0)),
        pl.BlockSpec((BR, D), lambda i: (i, 0)),
        pl.BlockSpec((D, D), lambda i: (0, 0)),
    ],
    out_specs=[pl.BlockSpec((BR, D), lambda i: (i, 0))] * 2,
    out_shape=[jax.ShapeDtypeStruct((N, D), jnp.float32)] * 2,
)


def _tc2_body(agg_ref, hs_ref, dinv_ref, b_ref, w_ref, out_ref):
    a = agg_ref[0] + agg_ref[1] + hs_ref[...]
    z = jnp.maximum(a * dinv_ref[...] + b_ref[...], 0.0)
    out_ref[...] = jnp.dot(
        z, w_ref[...], preferred_element_type=jnp.float32) * dinv_ref[...]


_tc2_call = pl.pallas_call(
    _tc2_body,
    grid=(N // BR,),
    in_specs=[
        pl.BlockSpec((NC, BR, D), lambda i: (0, i, 0)),
        pl.BlockSpec((BR, D), lambda i: (i, 0)),
        pl.BlockSpec((BR, D), lambda i: (i, 0)),
        pl.BlockSpec((1, D), lambda i: (0, 0)),
        pl.BlockSpec((D, D), lambda i: (0, 0)),
    ],
    out_specs=pl.BlockSpec((BR, D), lambda i: (i, 0)),
    out_shape=jax.ShapeDtypeStruct((N, D), jnp.float32),
)


def _tc3_body(agg_ref, hs_ref, dinv_ref, b_ref, out_ref):
    a = agg_ref[0] + agg_ref[1] + hs_ref[...]
    out_ref[...] = a * dinv_ref[...] + b_ref[...]


_tc3_call = pl.pallas_call(
    _tc3_body,
    grid=(N // BR,),
    in_specs=[
        pl.BlockSpec((NC, BR, D), lambda i: (0, i, 0)),
        pl.BlockSpec((BR, D), lambda i: (i, 0)),
        pl.BlockSpec((BR, D), lambda i: (i, 0)),
        pl.BlockSpec((1, D), lambda i: (0, 0)),
    ],
    out_specs=pl.BlockSpec((BR, D), lambda i: (i, 0)),
    out_shape=jax.ShapeDtypeStruct((N, D), jnp.float32),
)


@jax.jit
def kernel(x, edge_index, W1, b1, W2, b2):
    src = edge_index[0].astype(jnp.int32).reshape(NW, E // NW)
    dst = edge_index[1].astype(jnp.int32).reshape(NW, E // NW)
    ppt = (E2 - E) // NW
    pad_src = jnp.broadcast_to(
        (jnp.arange(ppt, dtype=jnp.int32) * 41) % N, (NW, ppt))
    pad_dst = jnp.broadcast_to(
        N + jnp.arange(ppt, dtype=jnp.int32), (NW, ppt))
    src3 = jnp.concatenate([src, pad_src], axis=1).reshape(NW, NCHUNK, CH)
    dst3 = jnp.concatenate([dst, pad_dst], axis=1).reshape(NW, NCHUNK, CH)
    deg_parts = _deg_call(dst3).reshape(NC, NPAD, 1)
    hs1, dinv = _tc1_call(deg_parts, x, W1)
    agg1 = _agg_call(src3, dst3, hs1).reshape(NC, NPAD, D)
    hs2 = _tc2_call(agg1, hs1, dinv, b1.reshape(1, D), W2)
    agg2 = _agg_call(src3, dst3, hs2).reshape(NC, NPAD, D)
    return _tc3_call(agg2, hs2, dinv, b2.reshape(1, D))


# speedup vs baseline: 1.2334x; 1.2334x over previous
import functools

import jax
import jax.numpy as jnp
from jax import lax
from jax.experimental import pallas as pl
from jax.experimental.pallas import tpu as pltpu
from jax.experimental.pallas import tpu_sc as plsc

N = 10000
E = 320000
D = 128
NC = 2
NS = 16
NW = NC * NS
CH = 128
NCHUNK = 80
E2 = NW * NCHUNK * CH
NH = NCHUNK // 2
NPAD = 10240
RPT = NPAD // NS
ZR = 128
DegW = 16
BR = 1000
L = 16


def _fill_const(ref, rows, width, value):
    @pl.loop(0, rows * (width // L))
    def _(i):
        r = i // (width // L)
        c = i % (width // L)
        ref[r, pl.ds(c * L, L)] = jnp.full((L,), value, jnp.float32)


def _deg_body(dst3_hbm, out_hbm, deg_sh, ones_v, idx_v, zbuf_v):
    c = lax.axis_index("c")
    s = lax.axis_index("s")
    wid = c * NS + s
    @pl.loop(0, (CH + L - 1) // L)
    def _(i):
        off = jnp.minimum(i * L, CH - L)
        ones_v[pl.ds(off, L)] = jnp.full((L,), 1.0, jnp.float32)
    @pl.loop(0, RPT // L)
    def _(i):
        zbuf_v[pl.ds(i * L, L)] = jnp.zeros((L,), jnp.float32)
    pltpu.sync_copy(zbuf_v, deg_sh.at[pl.ds(s * RPT, RPT)])
    pltpu.sync_copy(dst3_hbm.at[wid], idx_v)
    plsc.subcore_barrier()
    @pl.loop(0, NCHUNK)
    def _(k):
        pltpu.sync_copy(ones_v, deg_sh.at[idx_v.at[k]], add=True)
    plsc.subcore_barrier()
    pltpu.sync_copy(deg_sh.at[pl.ds(s * RPT, RPT)],
                    out_hbm.at[pl.ds(c * NPAD + s * RPT, RPT)])


def _agg_body(src3_hbm, dst3_hbm, tbl_hbm, out_hbm,
              acc_sh, sidx_v, didx_v, rows0, rows1, sg0, sg1):
    c = lax.axis_index("c")
    s = lax.axis_index("s")
    wid = c * NS + s
    _fill_const(rows0, CH, D, 0.0)
    @pl.loop(0, RPT // CH)
    def _(j):
        pltpu.sync_copy(rows0, acc_sh.at[pl.ds(s * RPT + j * CH, CH)])
    plsc.subcore_barrier()
    for h in range(2):
        pltpu.sync_copy(src3_hbm.at[wid, pl.ds(h * NH, NH)], sidx_v)
        pltpu.sync_copy(dst3_hbm.at[wid, pl.ds(h * NH, NH)], didx_v)
        pltpu.async_copy(tbl_hbm.at[sidx_v.at[0]], rows0, sg0)
        @pl.loop(0, NH - 2, step=2)
        def _(k):
            pltpu.async_copy(tbl_hbm.at[sidx_v.at[k + 1]], rows1, sg1)
            pltpu.make_async_copy(tbl_hbm.at[sidx_v.at[k]], rows0, sg0).wait()
            pltpu.sync_copy(rows0, acc_sh.at[didx_v.at[k]], add=True)
            pltpu.async_copy(tbl_hbm.at[sidx_v.at[k + 2]], rows0, sg0)
            pltpu.make_async_copy(tbl_hbm.at[sidx_v.at[k + 1]], rows1, sg1).wait()
            pltpu.sync_copy(rows1, acc_sh.at[didx_v.at[k + 1]], add=True)
        pltpu.async_copy(tbl_hbm.at[sidx_v.at[NH - 1]], rows1, sg1)
        pltpu.make_async_copy(tbl_hbm.at[sidx_v.at[NH - 2]], rows0, sg0).wait()
        pltpu.sync_copy(rows0, acc_sh.at[didx_v.at[NH - 2]], add=True)
        pltpu.make_async_copy(tbl_hbm.at[sidx_v.at[NH - 1]], rows1, sg1).wait()
        pltpu.sync_copy(rows1, acc_sh.at[didx_v.at[NH - 1]], add=True)
    plsc.subcore_barrier()
    pltpu.sync_copy(acc_sh.at[pl.ds(s * RPT, RPT)],
                    out_hbm.at[pl.ds(c * NPAD + s * RPT, RPT)])


_SC_MESH = plsc.VectorSubcoreMesh(
    core_axis_name="c", subcore_axis_name="s", num_cores=NC, num_subcores=NS)

_deg_call = pl.kernel(
    _deg_body,
    out_type=jax.ShapeDtypeStruct((NC * NPAD,), jnp.float32),
    mesh=_SC_MESH,
    scratch_types=[
        pltpu.VMEM_SHARED((NPAD,), jnp.float32),
        pltpu.VMEM((CH,), jnp.float32),
        pltpu.VMEM((NCHUNK, CH), jnp.int32),
        pltpu.VMEM((RPT,), jnp.float32),
    ],
)

_agg_call = pl.kernel(
    _agg_body,
    out_type=jax.ShapeDtypeStruct((NC * NPAD, D), jnp.float32),
    mesh=_SC_MESH,
    scratch_types=[
        pltpu.VMEM_SHARED((NPAD, D), jnp.float32),
        pltpu.VMEM((NH, CH), jnp.int32),
        pltpu.VMEM((NH, CH), jnp.int32),
        pltpu.VMEM((CH, D), jnp.float32),
        pltpu.VMEM((CH, D), jnp.float32),
        pltpu.SemaphoreType.DMA,
        pltpu.SemaphoreType.DMA,
    ],
)


def _tc1_body(deg_ref, x_ref, w_ref, hs_ref, dinv_ref):
    deg = deg_ref[0] + deg_ref[1] + 1.0
    dinv1 = lax.rsqrt(deg)
    h = jnp.dot(x_ref[...], w_ref[...], preferred_element_type=jnp.float32)
    hs_ref[...] = h * dinv1
    dinv_ref[...] = jnp.broadcast_to(dinv1, (BR, D))


_tc1_call = pl.pallas_call(
    _tc1_body,
    grid=(N // BR,),
    in_specs=[
        pl.BlockSpec((NC, BR, 1), lambda i: (0, i, 0)),
        pl.BlockSpec((BR, D), lambda i: (i, 0)),
        pl.BlockSpec((D, D), lambda i: (0, 0)),
    ],
    out_specs=[pl.BlockSpec((BR, D), lambda i: (i, 0))] * 2,
    out_shape=[jax.ShapeDtypeStruct((N, D), jnp.float32)] * 2,
)


def _tc2_body(agg_ref, hs_ref, dinv_ref, b_ref, w_ref, out_ref):
    a = agg_ref[0] + agg_ref[1] + hs_ref[...]
    z = jnp.maximum(a * dinv_ref[...] + b_ref[...], 0.0)
    out_ref[...] = jnp.dot(
        z, w_ref[...], preferred_element_type=jnp.float32) * dinv_ref[...]


_tc2_call = pl.pallas_call(
    _tc2_body,
    grid=(N // BR,),
    in_specs=[
        pl.BlockSpec((NC, BR, D), lambda i: (0, i, 0)),
        pl.BlockSpec((BR, D), lambda i: (i, 0)),
        pl.BlockSpec((BR, D), lambda i: (i, 0)),
        pl.BlockSpec((1, D), lambda i: (0, 0)),
        pl.BlockSpec((D, D), lambda i: (0, 0)),
    ],
    out_specs=pl.BlockSpec((BR, D), lambda i: (i, 0)),
    out_shape=jax.ShapeDtypeStruct((N, D), jnp.float32),
)


def _tc3_body(agg_ref, hs_ref, dinv_ref, b_ref, out_ref):
    a = agg_ref[0] + agg_ref[1] + hs_ref[...]
    out_ref[...] = a * dinv_ref[...] + b_ref[...]


_tc3_call = pl.pallas_call(
    _tc3_body,
    grid=(N // BR,),
    in_specs=[
        pl.BlockSpec((NC, BR, D), lambda i: (0, i, 0)),
        pl.BlockSpec((BR, D), lambda i: (i, 0)),
        pl.BlockSpec((BR, D), lambda i: (i, 0)),
        pl.BlockSpec((1, D), lambda i: (0, 0)),
    ],
    out_specs=pl.BlockSpec((BR, D), lambda i: (i, 0)),
    out_shape=jax.ShapeDtypeStruct((N, D), jnp.float32),
)


@jax.jit
def kernel(x, edge_index, W1, b1, W2, b2):
    src = edge_index[0].astype(jnp.int32).reshape(NW, E // NW)
    dst = edge_index[1].astype(jnp.int32).reshape(NW, E // NW)
    ppt = (E2 - E) // NW
    pad_src = jnp.broadcast_to(
        (jnp.arange(ppt, dtype=jnp.int32) * 41) % N, (NW, ppt))
    pad_dst = jnp.broadcast_to(
        N + jnp.arange(ppt, dtype=jnp.int32), (NW, ppt))
    src3 = jnp.concatenate([src, pad_src], axis=1).reshape(NW, NCHUNK, CH)
    dst3 = jnp.concatenate([dst, pad_dst], axis=1).reshape(NW, NCHUNK, CH)
    deg_parts = _deg_call(dst3).reshape(NC, NPAD, 1)
    hs1, dinv = _tc1_call(deg_parts, x, W1)
    agg1 = _agg_call(src3, dst3, hs1).reshape(NC, NPAD, D)
    hs2 = _tc2_call(agg1, hs1, dinv, b1.reshape(1, D), W2)
    agg2 = _agg_call(src3, dst3, hs2).reshape(NC, NPAD, D)
    return _tc3_call(agg2, hs2, dinv, b2.reshape(1, D))


# speedup vs baseline: 1.2804x; 1.0381x over previous
import functools

import numpy as np
import jax
import jax.numpy as jnp
from jax import lax
from jax.experimental import pallas as pl
from jax.experimental.pallas import tpu as pltpu
from jax.experimental.pallas import tpu_sc as plsc

N = 10000
E = 320000
D = 128
NC = 2
NS = 16
NW = NC * NS
CH = 128
NCHUNK = 80
E2 = NW * NCHUNK * CH
NH = NCHUNK // 2
NPAD = 10240
RPT = NPAD // NS
ZR = 128
DegW = 16
BR = 1000
BRM = 1024
L = 16

_PPT = (E2 - E) // NW
_PAD_SRC = jnp.asarray(
    np.broadcast_to((np.arange(_PPT) * 41) % N, (NW, _PPT)), jnp.int32)
_PAD_DST = jnp.asarray(
    np.broadcast_to(N + np.arange(_PPT), (NW, _PPT)), jnp.int32)


def _fill_const(ref, rows, width, value):
    @pl.loop(0, rows * (width // L))
    def _(i):
        r = i // (width // L)
        c = i % (width // L)
        ref[r, pl.ds(c * L, L)] = jnp.full((L,), value, jnp.float32)


def _deg_body(dst3_hbm, out_hbm, deg_sh, ones_v, idx_v, zbuf_v):
    c = lax.axis_index("c")
    s = lax.axis_index("s")
    wid = c * NS + s
    @pl.loop(0, (CH + L - 1) // L)
    def _(i):
        off = jnp.minimum(i * L, CH - L)
        ones_v[pl.ds(off, L)] = jnp.full((L,), 1.0, jnp.float32)
    @pl.loop(0, RPT // L)
    def _(i):
        zbuf_v[pl.ds(i * L, L)] = jnp.zeros((L,), jnp.float32)
    pltpu.sync_copy(zbuf_v, deg_sh.at[pl.ds(s * RPT, RPT)])
    pltpu.sync_copy(dst3_hbm.at[wid], idx_v)
    plsc.subcore_barrier()
    @pl.loop(0, NCHUNK)
    def _(k):
        pltpu.sync_copy(ones_v, deg_sh.at[idx_v.at[k]], add=True)
    plsc.subcore_barrier()
    pltpu.sync_copy(deg_sh.at[pl.ds(s * RPT, RPT)],
                    out_hbm.at[pl.ds(c * NPAD + s * RPT, RPT)])


def _agg_body(src3_hbm, dst3_hbm, tbl_hbm, out_hbm,
              acc_sh, sidx_v, didx_v, rows0, rows1, sg0, sg1):
    c = lax.axis_index("c")
    s = lax.axis_index("s")
    wid = c * NS + s
    _fill_const(rows0, CH, D, 0.0)
    @pl.loop(0, RPT // CH)
    def _(j):
        pltpu.sync_copy(rows0, acc_sh.at[pl.ds(s * RPT + j * CH, CH)])
    plsc.subcore_barrier()
    for h in range(2):
        pltpu.sync_copy(src3_hbm.at[wid, pl.ds(h * NH, NH)], sidx_v)
        pltpu.sync_copy(dst3_hbm.at[wid, pl.ds(h * NH, NH)], didx_v)
        pltpu.async_copy(tbl_hbm.at[sidx_v.at[0]], rows0, sg0)
        @pl.loop(0, NH - 2, step=2)
        def _(k):
            pltpu.async_copy(tbl_hbm.at[sidx_v.at[k + 1]], rows1, sg1)
            pltpu.make_async_copy(tbl_hbm.at[sidx_v.at[k]], rows0, sg0).wait()
            pltpu.sync_copy(rows0, acc_sh.at[didx_v.at[k]], add=True)
            pltpu.async_copy(tbl_hbm.at[sidx_v.at[k + 2]], rows0, sg0)
            pltpu.make_async_copy(tbl_hbm.at[sidx_v.at[k + 1]], rows1, sg1).wait()
            pltpu.sync_copy(rows1, acc_sh.at[didx_v.at[k + 1]], add=True)
        pltpu.async_copy(tbl_hbm.at[sidx_v.at[NH - 1]], rows1, sg1)
        pltpu.make_async_copy(tbl_hbm.at[sidx_v.at[NH - 2]], rows0, sg0).wait()
        pltpu.sync_copy(rows0, acc_sh.at[didx_v.at[NH - 2]], add=True)
        pltpu.make_async_copy(tbl_hbm.at[sidx_v.at[NH - 1]], rows1, sg1).wait()
        pltpu.sync_copy(rows1, acc_sh.at[didx_v.at[NH - 1]], add=True)
    plsc.subcore_barrier()
    pltpu.sync_copy(acc_sh.at[pl.ds(s * RPT, RPT)],
                    out_hbm.at[pl.ds(c * NPAD + s * RPT, RPT)])


_SC_MESH = plsc.VectorSubcoreMesh(
    core_axis_name="c", subcore_axis_name="s", num_cores=NC, num_subcores=NS)

_deg_call = pl.kernel(
    _deg_body,
    out_type=jax.ShapeDtypeStruct((NC * NPAD,), jnp.float32),
    mesh=_SC_MESH,
    scratch_types=[
        pltpu.VMEM_SHARED((NPAD,), jnp.float32),
        pltpu.VMEM((CH,), jnp.float32),
        pltpu.VMEM((NCHUNK, CH), jnp.int32),
        pltpu.VMEM((RPT,), jnp.float32),
    ],
)

_agg_call = pl.kernel(
    _agg_body,
    out_type=jax.ShapeDtypeStruct((NC * NPAD, D), jnp.float32),
    mesh=_SC_MESH,
    scratch_types=[
        pltpu.VMEM_SHARED((NPAD, D), jnp.float32),
        pltpu.VMEM((NH, CH), jnp.int32),
        pltpu.VMEM((NH, CH), jnp.int32),
        pltpu.VMEM((CH, D), jnp.float32),
        pltpu.VMEM((CH, D), jnp.float32),
        pltpu.SemaphoreType.DMA,
        pltpu.SemaphoreType.DMA,
    ],
)


def _tc1_body(deg_ref, x_ref, w_ref, hs_ref, dinv_ref):
    deg = deg_ref[0] + deg_ref[1] + 1.0
    dinv8 = lax.rsqrt(deg)
    ri = lax.broadcasted_iota(jnp.int32, (BRM, BRM // 128), 0) // 128
    ci = lax.broadcasted_iota(jnp.int32, (BRM, BRM // 128), 1)
    sel = (ri == ci).astype(jnp.float32)
    rep = jnp.dot(sel, dinv8, preferred_element_type=jnp.float32,
                  precision=lax.Precision.HIGHEST)
    rm = lax.broadcasted_iota(jnp.int32, (BRM, D), 0) % 128
    cm = lax.broadcasted_iota(jnp.int32, (BRM, D), 1)
    dinv_b = jnp.where(rm == cm, rep, 0.0)
    dinv1 = jnp.sum(dinv_b, axis=1, keepdims=True)
    h = jnp.dot(x_ref[...], w_ref[...], preferred_element_type=jnp.float32)
    hs_ref[...] = h * dinv1
    dinv_ref[...] = jnp.broadcast_to(dinv1, (BRM, D))


_tc1_call = pl.pallas_call(
    _tc1_body,
    grid=(NPAD // BRM,),
    in_specs=[
        pl.BlockSpec((NC, BRM // 128, 128), lambda i: (0, i, 0)),
        pl.BlockSpec((BRM, D), lambda i: (i, 0)),
        pl.BlockSpec((D, D), lambda i: (0, 0)),
    ],
    out_specs=[pl.BlockSpec((BRM, D), lambda i: (i, 0))] * 2,
    out_shape=[jax.ShapeDtypeStruct((N, D), jnp.float32)] * 2,
)


def _tc2_body(agg_ref, hs_ref, dinv_ref, b_ref, w_ref, out_ref):
    a = agg_ref[0] + agg_ref[1] + hs_ref[...]
    z = jnp.maximum(a * dinv_ref[...] + b_ref[...], 0.0)
    out_ref[...] = jnp.dot(
        z, w_ref[...], preferred_element_type=jnp.float32) * dinv_ref[...]


_tc2_call = pl.pallas_call(
    _tc2_body,
    grid=(N // BR,),
    in_specs=[
        pl.BlockSpec((NC, BR, D), lambda i: (0, i, 0)),
        pl.BlockSpec((BR, D), lambda i: (i, 0)),
        pl.BlockSpec((BR, D), lambda i: (i, 0)),
        pl.BlockSpec((1, D), lambda i: (0, 0)),
        pl.BlockSpec((D, D), lambda i: (0, 0)),
    ],
    out_specs=pl.BlockSpec((BR, D), lambda i: (i, 0)),
    out_shape=jax.ShapeDtypeStruct((N, D), jnp.float32),
)


def _tc3_body(agg_ref, hs_ref, dinv_ref, b_ref, out_ref):
    a = agg_ref[0] + agg_ref[1] + hs_ref[...]
    out_ref[...] = a * dinv_ref[...] + b_ref[...]


_tc3_call = pl.pallas_call(
    _tc3_body,
    grid=(N // BR,),
    in_specs=[
        pl.BlockSpec((NC, BR, D), lambda i: (0, i, 0)),
        pl.BlockSpec((BR, D), lambda i: (i, 0)),
        pl.BlockSpec((BR, D), lambda i: (i, 0)),
        pl.BlockSpec((1, D), lambda i: (0, 0)),
    ],
    out_specs=pl.BlockSpec((BR, D), lambda i: (i, 0)),
    out_shape=jax.ShapeDtypeStruct((N, D), jnp.float32),
)


@jax.jit
def kernel(x, edge_index, W1, b1, W2, b2):
    src = edge_index[0].astype(jnp.int32).reshape(NW, E // NW)
    dst = edge_index[1].astype(jnp.int32).reshape(NW, E // NW)
    src3 = jnp.concatenate([src, _PAD_SRC], axis=1).reshape(NW, NCHUNK, CH)
    dst3 = jnp.concatenate([dst, _PAD_DST], axis=1).reshape(NW, NCHUNK, CH)
    deg_parts = _deg_call(dst3).reshape(NC, NPAD // 128, 128)
    hs1, dinv = _tc1_call(deg_parts, x, W1)
    agg1 = _agg_call(src3, dst3, hs1).reshape(NC, NPAD, D)
    hs2 = _tc2_call(agg1, hs1, dinv, b1.reshape(1, D), W2)
    agg2 = _agg_call(src3, dst3, hs2).reshape(NC, NPAD, D)
    return _tc3_call(agg2, hs2, dinv, b2.reshape(1, D))


# speedup vs baseline: 1.2808x; 1.0003x over previous
import functools

import numpy as np
import jax
import jax.numpy as jnp
from jax import lax
from jax.experimental import pallas as pl
from jax.experimental.pallas import tpu as pltpu
from jax.experimental.pallas import tpu_sc as plsc

N = 10000
E = 320000
D = 128
NC = 2
NS = 16
NW = NC * NS
CH = 128
NCHUNK = 80
E2 = NW * NCHUNK * CH
NH = NCHUNK // 2
NPAD = 10240
RPT = NPAD // NS
ZR = 128
DegW = 16
BR = 1000
BRM = 1024
L = 16

_PPT = (E2 - E) // NW
_PAD_SRC = jnp.asarray(
    np.broadcast_to((np.arange(_PPT) * 41) % N, (NW, _PPT)), jnp.int32)
_PAD_DST = jnp.asarray(
    np.broadcast_to(N + np.arange(_PPT), (NW, _PPT)), jnp.int32)


def _fill_const(ref, rows, width, value):
    @pl.loop(0, rows * (width // L))
    def _(i):
        r = i // (width // L)
        c = i % (width // L)
        ref[r, pl.ds(c * L, L)] = jnp.full((L,), value, jnp.float32)


def _deg_body(dst3_hbm, out_hbm, deg_sh, ones_v, idx_v, zbuf_v):
    c = lax.axis_index("c")
    s = lax.axis_index("s")
    wid = c * NS + s
    @pl.loop(0, (CH + L - 1) // L)
    def _(i):
        off = jnp.minimum(i * L, CH - L)
        ones_v[pl.ds(off, L)] = jnp.full((L,), 1.0, jnp.float32)
    @pl.loop(0, RPT // L)
    def _(i):
        zbuf_v[pl.ds(i * L, L)] = jnp.zeros((L,), jnp.float32)
    pltpu.sync_copy(zbuf_v, deg_sh.at[pl.ds(s * RPT, RPT)])
    pltpu.sync_copy(dst3_hbm.at[wid], idx_v)
    plsc.subcore_barrier()
    @pl.loop(0, NCHUNK)
    def _(k):
        pltpu.sync_copy(ones_v, deg_sh.at[idx_v.at[k]], add=True)
    plsc.subcore_barrier()
    pltpu.sync_copy(deg_sh.at[pl.ds(s * RPT, RPT)],
                    out_hbm.at[pl.ds(c * NPAD + s * RPT, RPT)])


def _agg_body(src3_hbm, dst3_hbm, tbl_hbm, out_hbm,
              acc_sh, sidx_v, didx_v, rows0, rows1, sg0, sg1):
    c = lax.axis_index("c")
    s = lax.axis_index("s")
    wid = c * NS + s
    _fill_const(rows0, CH, D, 0.0)
    @pl.loop(0, RPT // CH)
    def _(j):
        pltpu.sync_copy(rows0, acc_sh.at[pl.ds(s * RPT + j * CH, CH)])
    plsc.subcore_barrier()
    for h in range(2):
        pltpu.sync_copy(src3_hbm.at[wid, pl.ds(h * NH, NH)], sidx_v)
        pltpu.sync_copy(dst3_hbm.at[wid, pl.ds(h * NH, NH)], didx_v)
        pltpu.async_copy(tbl_hbm.at[sidx_v.at[0]], rows0, sg0)
        @pl.loop(0, NH - 2, step=2)
        def _(k):
            pltpu.async_copy(tbl_hbm.at[sidx_v.at[k + 1]], rows1, sg1)
            pltpu.make_async_copy(tbl_hbm.at[sidx_v.at[k]], rows0, sg0).wait()
            pltpu.sync_copy(rows0, acc_sh.at[didx_v.at[k]], add=True)
            pltpu.async_copy(tbl_hbm.at[sidx_v.at[k + 2]], rows0, sg0)
            pltpu.make_async_copy(tbl_hbm.at[sidx_v.at[k + 1]], rows1, sg1).wait()
            pltpu.sync_copy(rows1, acc_sh.at[didx_v.at[k + 1]], add=True)
        pltpu.async_copy(tbl_hbm.at[sidx_v.at[NH - 1]], rows1, sg1)
        pltpu.make_async_copy(tbl_hbm.at[sidx_v.at[NH - 2]], rows0, sg0).wait()
        pltpu.sync_copy(rows0, acc_sh.at[didx_v.at[NH - 2]], add=True)
        pltpu.make_async_copy(tbl_hbm.at[sidx_v.at[NH - 1]], rows1, sg1).wait()
        pltpu.sync_copy(rows1, acc_sh.at[didx_v.at[NH - 1]], add=True)
    plsc.subcore_barrier()
    pltpu.sync_copy(acc_sh.at[pl.ds(s * RPT, RPT)],
                    out_hbm.at[pl.ds(c * NPAD + s * RPT, RPT)])


_SC_MESH = plsc.VectorSubcoreMesh(
    core_axis_name="c", subcore_axis_name="s", num_cores=NC, num_subcores=NS)

_deg_call = pl.kernel(
    _deg_body,
    out_type=jax.ShapeDtypeStruct((NC * NPAD,), jnp.float32),
    mesh=_SC_MESH,
    scratch_types=[
        pltpu.VMEM_SHARED((NPAD,), jnp.float32),
        pltpu.VMEM((CH,), jnp.float32),
        pltpu.VMEM((NCHUNK, CH), jnp.int32),
        pltpu.VMEM((RPT,), jnp.float32),
    ],
)

_agg_call = pl.kernel(
    _agg_body,
    out_type=jax.ShapeDtypeStruct((NC * NPAD, D), jnp.float32),
    mesh=_SC_MESH,
    scratch_types=[
        pltpu.VMEM_SHARED((NPAD, D), jnp.float32),
        pltpu.VMEM((NH, CH), jnp.int32),
        pltpu.VMEM((NH, CH), jnp.int32),
        pltpu.VMEM((CH, D), jnp.float32),
        pltpu.VMEM((CH, D), jnp.float32),
        pltpu.SemaphoreType.DMA,
        pltpu.SemaphoreType.DMA,
    ],
)


def _dinv_col(deg_ref):
    deg = deg_ref[0] + deg_ref[1] + 1.0
    dinv8 = lax.rsqrt(deg)
    ri = lax.broadcasted_iota(jnp.int32, (BRM, BRM // 128), 0) // 128
    ci = lax.broadcasted_iota(jnp.int32, (BRM, BRM // 128), 1)
    sel = (ri == ci).astype(jnp.float32)
    rep = jnp.dot(sel, dinv8, preferred_element_type=jnp.float32,
                  precision=lax.Precision.HIGHEST)
    rm = lax.broadcasted_iota(jnp.int32, (BRM, D), 0) % 128
    cm = lax.broadcasted_iota(jnp.int32, (BRM, D), 1)
    return jnp.sum(jnp.where(rm == cm, rep, 0.0), axis=1, keepdims=True)


def _tc1_body(deg_ref, x_ref, w_ref, hs_ref):
    dinv1 = _dinv_col(deg_ref)
    h = jnp.dot(x_ref[...], w_ref[...], preferred_element_type=jnp.float32)
    hs_ref[...] = h * dinv1


_tc1_call = pl.pallas_call(
    _tc1_body,
    grid=(NPAD // BRM,),
    in_specs=[
        pl.BlockSpec((NC, BRM // 128, 128), lambda i: (0, i, 0)),
        pl.BlockSpec((BRM, D), lambda i: (i, 0)),
        pl.BlockSpec((D, D), lambda i: (0, 0)),
    ],
    out_specs=pl.BlockSpec((BRM, D), lambda i: (i, 0)),
    out_shape=jax.ShapeDtypeStruct((N, D), jnp.float32),
)


def _tc2_body(agg_ref, hs_ref, deg_ref, b_ref, w_ref, out_ref):
    dinv1 = _dinv_col(deg_ref)
    a = agg_ref[0] + agg_ref[1] + hs_ref[...]
    z = jnp.maximum(a * dinv1 + b_ref[...], 0.0)
    out_ref[...] = jnp.dot(
        z, w_ref[...], preferred_element_type=jnp.float32) * dinv1


_tc2_call = pl.pallas_call(
    _tc2_body,
    grid=(NPAD // BRM,),
    in_specs=[
        pl.BlockSpec((NC, BRM, D), lambda i: (0, i, 0)),
        pl.BlockSpec((BRM, D), lambda i: (i, 0)),
        pl.BlockSpec((NC, BRM // 128, 128), lambda i: (0, i, 0)),
        pl.BlockSpec((1, D), lambda i: (0, 0)),
        pl.BlockSpec((D, D), lambda i: (0, 0)),
    ],
    out_specs=pl.BlockSpec((BRM, D), lambda i: (i, 0)),
    out_shape=jax.ShapeDtypeStruct((N, D), jnp.float32),
)


def _tc3_body(agg_ref, hs_ref, deg_ref, b_ref, out_ref):
    dinv1 = _dinv_col(deg_ref)
    a = agg_ref[0] + agg_ref[1] + hs_ref[...]
    out_ref[...] = a * dinv1 + b_ref[...]


_tc3_call = pl.pallas_call(
    _tc3_body,
    grid=(NPAD // BRM,),
    in_specs=[
        pl.BlockSpec((NC, BRM, D), lambda i: (0, i, 0)),
        pl.BlockSpec((BRM, D), lambda i: (i, 0)),
        pl.BlockSpec((NC, BRM // 128, 128), lambda i: (0, i, 0)),
        pl.BlockSpec((1, D), lambda i: (0, 0)),
    ],
    out_specs=pl.BlockSpec((BRM, D), lambda i: (i, 0)),
    out_shape=jax.ShapeDtypeStruct((N, D), jnp.float32),
)


@jax.jit
def kernel(x, edge_index, W1, b1, W2, b2):
    es = edge_index.astype(jnp.int32).reshape(2, NW, E // NW)
    src = es[0]
    dst = es[1]
    src3 = jnp.concatenate([src, _PAD_SRC], axis=1).reshape(NW, NCHUNK, CH)
    dst3 = jnp.concatenate([dst, _PAD_DST], axis=1).reshape(NW, NCHUNK, CH)
    deg3 = _deg_call(dst3).reshape(NC, NPAD // 128, 128)
    hs1 = _tc1_call(deg3, x, W1)
    agg1 = _agg_call(src3, dst3, hs1).reshape(NC, NPAD, D)
    hs2 = _tc2_call(agg1, hs1, deg3, b1.reshape(1, D), W2)
    agg2 = _agg_call(src3, dst3, hs2).reshape(NC, NPAD, D)
    return _tc3_call(agg2, hs2, deg3, b2.reshape(1, D))
